# Initial kernel scaffold; baseline (speedup 1.0000x reference)
import jax
import jax.numpy as jnp
from jax.experimental import pallas as pl


def kernel(x, edge_index, batch, W1, b1, W2, b2, Wc, bc):
    raise NotImplementedError("write your pallas kernel here")


# speedup vs baseline: 16.4941x; 16.4941x over previous
import functools

import jax
import jax.numpy as jnp
from jax import lax
from jax.experimental import pallas as pl
from jax.experimental.pallas import tpu as pltpu
from jax.experimental.pallas import tpu_sc as plsc

N_NODES = 10000
NP = 10240
E = 320000
CHUNK = 128
EROWS = 2560
EP = EROWS * CHUNK
NW = 32
ROWS_PER_W = EROWS // NW
D_IN = 128
DH = 64
G = 64
GP = G + 1
SEG_ROWS = NP // NW
ACC_ROWS = NP // 16
NEG = -3.4028235e38


def _mesh():
    return plsc.VectorSubcoreMesh(
        core_axis_name="c", subcore_axis_name="s", num_cores=2, num_subcores=16
    )


def _sc_deg(dst2d):
    @functools.partial(
        pl.kernel,
        out_type=jax.ShapeDtypeStruct((NW, NP), jnp.float32),
        mesh=_mesh(),
        scratch_types=[
            pltpu.VMEM((CHUNK,), jnp.int32),
            pltpu.VMEM((NP,), jnp.float32),
        ],
        compiler_params=pltpu.CompilerParams(needs_layout_passes=False),
    )
    def k(dst_hbm, deg_hbm, idx_v, deg_v):
        c = lax.axis_index("c")
        s = lax.axis_index("s")
        w = s * 2 + c
        z16 = jnp.zeros((16,), jnp.float32)

        def zero_body(i, _):
            deg_v[pl.ds(i * 16, 16)] = z16
            return 0

        lax.fori_loop(0, NP // 16, zero_body, 0)

        def row_body(r, _):
            pltpu.sync_copy(dst_hbm.at[w * ROWS_PER_W + r], idx_v)

            def vec_body(j, _):
                idx = idx_v[pl.ds(j * 16, 16)]
                cnt, last = plsc.scan_count(idx)
                plsc.addupdate_scatter(
                    deg_v, [idx], cnt.astype(jnp.float32), mask=last
                )
                return 0

            lax.fori_loop(0, CHUNK // 16, vec_body, 0)
            return 0

        lax.fori_loop(0, ROWS_PER_W, row_body, 0)
        pltpu.sync_copy(deg_v, deg_hbm.at[w])

    return k(dst2d)


def _sc_agg(h, src2d, dst2d):
    @functools.partial(
        pl.kernel,
        out_type=jax.ShapeDtypeStruct((2, NP, DH), jnp.float32),
        mesh=_mesh(),
        scratch_types=[
            pltpu.VMEM((ROWS_PER_W, CHUNK), jnp.int32),
            pltpu.VMEM((ROWS_PER_W, CHUNK), jnp.int32),
            pltpu.VMEM((2, CHUNK, DH), jnp.float32),
            pltpu.VMEM((CHUNK, DH), jnp.float32),
            pltpu.VMEM_SHARED((NP, DH), jnp.float32),
            pltpu.SemaphoreType.DMA,
            pltpu.SemaphoreType.DMA,
        ],
        compiler_params=pltpu.CompilerParams(use_tc_tiling_on_sc=False),
    )
    def k(h_hbm, src_hbm, dst_hbm, out_hbm, src_v, dst_v, gbuf, zbuf, acc_sh,
          sem0, sem1):
        c = lax.axis_index("c")
        s = lax.axis_index("s")
        w = s * 2 + c
        z16 = jnp.zeros((16,), jnp.float32)

        def zero_body(i, _):
            for jj in range(DH // 16):
                zbuf[i, pl.ds(jj * 16, 16)] = z16
            return 0

        lax.fori_loop(0, CHUNK, zero_body, 0)
        for t in range(ACC_ROWS // CHUNK):
            pltpu.sync_copy(zbuf, acc_sh.at[pl.ds(s * ACC_ROWS + t * CHUNK, CHUNK)])
        plsc.subcore_barrier()

        pltpu.sync_copy(src_hbm.at[pl.ds(w * ROWS_PER_W, ROWS_PER_W)], src_v)
        pltpu.sync_copy(dst_hbm.at[pl.ds(w * ROWS_PER_W, ROWS_PER_W)], dst_v)

        sems = (sem0, sem1)
        pltpu.async_copy(h_hbm.at[src_v.at[0]], gbuf.at[0], sem0)

        def pair_body(i, _):
            for par in range(2):
                r = 2 * i + par

                @pl.when(r + 1 < ROWS_PER_W)
                def _(r=r, par=par):
                    pltpu.async_copy(
                        h_hbm.at[src_v.at[r + 1]], gbuf.at[1 - par],
                        sems[1 - par],
                    )

                @pl.when(r < ROWS_PER_W)
                def _(r=r, par=par):
                    pltpu.make_async_copy(
                        h_hbm.at[src_v.at[r]], gbuf.at[par], sems[par]
                    ).wait()
                    pltpu.sync_copy(
                        gbuf.at[par], acc_sh.at[dst_v.at[r]], add=True
                    )
            return 0

        lax.fori_loop(0, (ROWS_PER_W + 1) // 2, pair_body, 0)
        plsc.subcore_barrier()
        pltpu.sync_copy(
            acc_sh.at[pl.ds(s * ACC_ROWS, ACC_ROWS)],
            out_hbm.at[c, pl.ds(s * ACC_ROWS, ACC_ROWS)],
        )

    return k(h, src2d, dst2d)


def _sc_segmax(p2, h2s, dinv1d, b2, batch_p):
    @functools.partial(
        pl.kernel,
        out_type=jax.ShapeDtypeStruct((NW, GP * DH), jnp.float32),
        mesh=_mesh(),
        scratch_types=[
            pltpu.VMEM((SEG_ROWS, DH), jnp.float32),
            pltpu.VMEM((SEG_ROWS, DH), jnp.float32),
            pltpu.VMEM((SEG_ROWS, DH), jnp.float32),
            pltpu.VMEM((SEG_ROWS,), jnp.float32),
            pltpu.VMEM((SEG_ROWS,), jnp.int32),
            pltpu.VMEM((DH,), jnp.float32),
            pltpu.VMEM((GP * DH,), jnp.float32),
        ],
        compiler_params=pltpu.CompilerParams(needs_layout_passes=False),
    )
    def k(p_hbm, h_hbm, dinv_hbm, b2_hbm, batch_hbm, out_hbm,
          p0v, p1v, hv, dv, bv, b2v, pooled):
        c = lax.axis_index("c")
        s = lax.axis_index("s")
        w = s * 2 + c
        base = w * SEG_ROWS
        pltpu.sync_copy(p_hbm.at[0, pl.ds(base, SEG_ROWS)], p0v)
        pltpu.sync_copy(p_hbm.at[1, pl.ds(base, SEG_ROWS)], p1v)
        pltpu.sync_copy(h_hbm.at[pl.ds(base, SEG_ROWS)], hv)
        pltpu.sync_copy(dinv_hbm.at[pl.ds(base, SEG_ROWS)], dv)
        pltpu.sync_copy(batch_hbm.at[pl.ds(base, SEG_ROWS)], bv)
        pltpu.sync_copy(b2_hbm, b2v)

        neg16 = jnp.full((16,), NEG, jnp.float32)

        def init_body(i, _):
            pooled[pl.ds(i * 16, 16)] = neg16
            return 0

        lax.fori_loop(0, GP * DH // 16, init_body, 0)

        iota16 = lax.iota(jnp.int32, 16)

        def row_body(n, _):
            nvec = jnp.full((16,), n, jnp.int32)
            bvec = plsc.load_gather(bv, [nvec])
            dvec = plsc.load_gather(dv, [nvec])
            pbase = bvec * DH + iota16
            for j in range(DH // 16):
                sl = pl.ds(j * 16, 16)
                val = dvec * (
                    p0v[n, sl] + p1v[n, sl] + hv[n, sl]
                ) + b2v[sl]
                idx = pbase + j * 16
                cur = plsc.load_gather(pooled, [idx])
                plsc.store_scatter(pooled, [idx], jnp.maximum(cur, val))
            return 0

        lax.fori_loop(0, SEG_ROWS, row_body, 0)
        pltpu.sync_copy(pooled, out_hbm.at[w])

    return k(
        p2.reshape(2, NP * DH).reshape(2, NP, DH),
        h2s,
        dinv1d,
        b2,
        batch_p,
    )


def _tc1_body(x_ref, w_ref, deg_ref, h_ref, dinv_ref):
    deg = jnp.sum(deg_ref[...], axis=0) + 1.0
    dinv = lax.rsqrt(deg)
    h = jnp.dot(x_ref[...], w_ref[...], preferred_element_type=jnp.float32)
    h_ref[...] = h * dinv[:, None]
    dinv_ref[...] = dinv[:, None]


def _tc1(xp, W1, degp):
    bn = 1024
    return pl.pallas_call(
        _tc1_body,
        grid=(NP // bn,),
        in_specs=[
            pl.BlockSpec((bn, D_IN), lambda i: (i, 0)),
            pl.BlockSpec((D_IN, DH), lambda i: (0, 0)),
            pl.BlockSpec((NW, bn), lambda i: (0, i)),
        ],
        out_specs=[
            pl.BlockSpec((bn, DH), lambda i: (i, 0)),
            pl.BlockSpec((bn, 1), lambda i: (i, 0)),
        ],
        out_shape=[
            jax.ShapeDtypeStruct((NP, DH), jnp.float32),
            jax.ShapeDtypeStruct((NP, 1), jnp.float32),
        ],
    )(xp, W1, degp)


def _tc2_body(p_ref, h_ref, dinv_ref, b1_ref, w2_ref, out_ref):
    acc = p_ref[0] + p_ref[1]
    dinv = dinv_ref[...]
    t = jnp.maximum(dinv * (acc + h_ref[...]) + b1_ref[...], 0.0)
    out_ref[...] = (
        jnp.dot(t, w2_ref[...], preferred_element_type=jnp.float32) * dinv
    )


def _tc2(p1, h1s, dinv, b1row, W2):
    bn = 1024
    return pl.pallas_call(
        _tc2_body,
        grid=(NP // bn,),
        in_specs=[
            pl.BlockSpec((2, bn, DH), lambda i: (0, i, 0)),
            pl.BlockSpec((bn, DH), lambda i: (i, 0)),
            pl.BlockSpec((bn, 1), lambda i: (i, 0)),
            pl.BlockSpec((1, DH), lambda i: (0, 0)),
            pl.BlockSpec((DH, DH), lambda i: (0, 0)),
        ],
        out_specs=pl.BlockSpec((bn, DH), lambda i: (i, 0)),
        out_shape=jax.ShapeDtypeStruct((NP, DH), jnp.float32),
    )(p1, h1s, dinv, b1row, W2)


def _tc3_body(part_ref, wc_ref, bc_ref, out_ref):
    parts = part_ref[...]
    pooled = jnp.max(parts[:, :G, :], axis=0)
    logits = (
        jnp.dot(pooled, wc_ref[...], preferred_element_type=jnp.float32)
        + bc_ref[...]
    )
    m = jnp.max(logits, axis=1, keepdims=True)
    lse = jnp.log(jnp.sum(jnp.exp(logits - m), axis=1, keepdims=True)) + m
    out_ref[...] = logits - lse


def _tc3(parts, Wc, bcrow):
    nc = bcrow.shape[-1]
    return pl.pallas_call(
        _tc3_body,
        in_specs=[
            pl.BlockSpec((NW, GP, DH), lambda: (0, 0, 0)),
            pl.BlockSpec((DH, nc), lambda: (0, 0)),
            pl.BlockSpec((1, nc), lambda: (0, 0)),
        ],
        out_specs=pl.BlockSpec((G, nc), lambda: (0, 0)),
        out_shape=jax.ShapeDtypeStruct((G, nc), jnp.float32),
    )(parts, Wc, bcrow)


def kernel(x, edge_index, batch, W1, b1, W2, b2, Wc, bc):
    src = edge_index[0].astype(jnp.int32)
    dst = edge_index[1].astype(jnp.int32)
    pad = jnp.full((EP - E,), N_NODES, jnp.int32)
    src2d = jnp.concatenate([src, pad]).reshape(EROWS, CHUNK)
    dst2d = jnp.concatenate([dst, pad]).reshape(EROWS, CHUNK)
    xp = jnp.pad(x, ((0, NP - N_NODES), (0, 0)))
    batch_p = jnp.concatenate(
        [batch.astype(jnp.int32), jnp.full((NP - N_NODES,), G, jnp.int32)]
    )

    degp = _sc_deg(dst2d)
    h1s, dinv = _tc1(xp, W1, degp)
    p1 = _sc_agg(h1s, src2d, dst2d)
    h2s = _tc2(p1, h1s, dinv, b1.reshape(1, DH), W2)
    p2 = _sc_agg(h2s, src2d, dst2d)
    parts = _sc_segmax(p2, h2s, dinv.reshape(NP), b2, batch_p)
    return _tc3(parts.reshape(NW, GP, DH), Wc, bc.reshape(1, -1))


# speedup vs baseline: 17.8964x; 1.0850x over previous
import functools

import jax
import jax.numpy as jnp
from jax import lax
from jax.experimental import pallas as pl
from jax.experimental.pallas import tpu as pltpu
from jax.experimental.pallas import tpu_sc as plsc

N_NODES = 10000
NP = 10240
E = 320000
CHUNK = 128
EROWS = 2560
EP = EROWS * CHUNK
NW = 32
ROWS_PER_W = EROWS // NW
NB = 8
LOOKAHEAD = 4
D_IN = 128
DH = 64
G = 64
GP = G + 1
SEG_ROWS = NP // NW
ACC_ROWS = NP // 16
NEG = -3.4028235e38


def _mesh():
    return plsc.VectorSubcoreMesh(
        core_axis_name="c", subcore_axis_name="s", num_cores=2, num_subcores=16
    )


def _sc_deg(dst2d):
    @functools.partial(
        pl.kernel,
        out_type=jax.ShapeDtypeStruct((NW, NP), jnp.float32),
        mesh=_mesh(),
        scratch_types=[
            pltpu.VMEM((ROWS_PER_W, CHUNK), jnp.int32),
            pltpu.VMEM((NP,), jnp.float32),
            pltpu.SemaphoreType.DMA,
        ],
        compiler_params=pltpu.CompilerParams(needs_layout_passes=False),
    )
    def k(dst_hbm, deg_hbm, idx_v, deg_v, sem):
        c = lax.axis_index("c")
        s = lax.axis_index("s")
        w = s * 2 + c
        cp = pltpu.async_copy(
            dst_hbm.at[pl.ds(w * ROWS_PER_W, ROWS_PER_W)], idx_v, sem
        )
        z16 = jnp.zeros((16,), jnp.float32)

        def zero_body(i, _):
            deg_v[pl.ds(i * 16, 16)] = z16
            return 0

        lax.fori_loop(0, NP // 16, zero_body, 0)
        cp.wait()

        def vec_body(j, _):
            idx = idx_v[j // (CHUNK // 16), pl.ds((j % (CHUNK // 16)) * 16, 16)]
            cnt, last = plsc.scan_count(idx)
            plsc.addupdate_scatter(
                deg_v, [idx], cnt.astype(jnp.float32), mask=last
            )
            return 0

        lax.fori_loop(0, ROWS_PER_W * (CHUNK // 16), vec_body, 0)
        pltpu.sync_copy(deg_v, deg_hbm.at[w])

    return k(dst2d)


def _sc_agg(h, src2d, dst2d):
    @functools.partial(
        pl.kernel,
        out_type=jax.ShapeDtypeStruct((2, NP, DH), jnp.float32),
        mesh=_mesh(),
        scratch_types=[
            pltpu.VMEM((ROWS_PER_W, CHUNK), jnp.int32),
            pltpu.VMEM((ROWS_PER_W, CHUNK), jnp.int32),
            pltpu.VMEM((NB, CHUNK, DH), jnp.float32),
            pltpu.VMEM_SHARED((NP, DH), jnp.float32),
            pltpu.SemaphoreType.DMA((NB,)),
            pltpu.SemaphoreType.DMA((NB,)),
            pltpu.SemaphoreType.DMA((2,)),
        ],
        compiler_params=pltpu.CompilerParams(use_tc_tiling_on_sc=False),
    )
    def k(h_hbm, src_hbm, dst_hbm, out_hbm, src_v, dst_v, gbuf, acc_sh,
          sem_g, sem_s, sem_i):
        c = lax.axis_index("c")
        s = lax.axis_index("s")
        w = s * 2 + c
        cps = pltpu.async_copy(
            src_hbm.at[pl.ds(w * ROWS_PER_W, ROWS_PER_W)], src_v, sem_i.at[0]
        )
        cpd = pltpu.async_copy(
            dst_hbm.at[pl.ds(w * ROWS_PER_W, ROWS_PER_W)], dst_v, sem_i.at[1]
        )
        z16 = jnp.zeros((16,), jnp.float32)

        def zero_body(i, _):
            for jj in range(DH // 16):
                gbuf[0, i, pl.ds(jj * 16, 16)] = z16
            return 0

        lax.fori_loop(0, CHUNK, zero_body, 0)
        for t in range(ACC_ROWS // CHUNK):
            pltpu.sync_copy(
                gbuf.at[0], acc_sh.at[pl.ds(s * ACC_ROWS + t * CHUNK, CHUNK)]
            )
        cps.wait()
        cpd.wait()
        plsc.subcore_barrier()

        for p in range(LOOKAHEAD):
            pltpu.async_copy(h_hbm.at[src_v.at[p]], gbuf.at[p], sem_g.at[p])

        def round_body(r, _):
            for par in range(NB):
                cidx = r * NB + par
                pltpu.make_async_copy(
                    h_hbm.at[src_v.at[cidx]], gbuf.at[par], sem_g.at[par]
                ).wait()
                pltpu.async_copy(
                    gbuf.at[par], acc_sh.at[dst_v.at[cidx]], sem_s.at[par],
                    add=True,
                )
                nslot = (par + LOOKAHEAD) % NB
                nxt = cidx + LOOKAHEAD

                @pl.when(nxt >= NB)
                def _(nxt=nxt, nslot=nslot):
                    pltpu.make_async_copy(
                        gbuf.at[nslot], acc_sh.at[dst_v.at[nxt - NB]],
                        sem_s.at[nslot],
                    ).wait()

                @pl.when(nxt < ROWS_PER_W)
                def _(nxt=nxt, nslot=nslot):
                    pltpu.async_copy(
                        h_hbm.at[src_v.at[nxt]], gbuf.at[nslot],
                        sem_g.at[nslot],
                    )
            return 0

        lax.fori_loop(0, ROWS_PER_W // NB, round_body, 0)
        for p in range(NB - LOOKAHEAD, NB):
            pltpu.make_async_copy(
                gbuf.at[p],
                acc_sh.at[dst_v.at[ROWS_PER_W - NB + p]],
                sem_s.at[p],
            ).wait()
        plsc.subcore_barrier()
        pltpu.sync_copy(
            acc_sh.at[pl.ds(s * ACC_ROWS, ACC_ROWS)],
            out_hbm.at[c, pl.ds(s * ACC_ROWS, ACC_ROWS)],
        )

    return k(h, src2d, dst2d)


def _sc_segmax(p2, h2s, dinv1d, b2, batch_p):
    @functools.partial(
        pl.kernel,
        out_type=jax.ShapeDtypeStruct((NW, GP * DH), jnp.float32),
        mesh=_mesh(),
        scratch_types=[
            pltpu.VMEM((SEG_ROWS, DH), jnp.float32),
            pltpu.VMEM((SEG_ROWS, DH), jnp.float32),
            pltpu.VMEM((SEG_ROWS, DH), jnp.float32),
            pltpu.VMEM((SEG_ROWS,), jnp.float32),
            pltpu.VMEM((SEG_ROWS,), jnp.int32),
            pltpu.VMEM((DH,), jnp.float32),
            pltpu.VMEM((GP * DH,), jnp.float32),
            pltpu.SemaphoreType.DMA((6,)),
        ],
        compiler_params=pltpu.CompilerParams(needs_layout_passes=False),
    )
    def k(p_hbm, h_hbm, dinv_hbm, b2_hbm, batch_hbm, out_hbm,
          p0v, p1v, hv, dv, bv, b2v, pooled, sems):
        c = lax.axis_index("c")
        s = lax.axis_index("s")
        w = s * 2 + c
        base = w * SEG_ROWS
        cps = [
            pltpu.async_copy(p_hbm.at[0, pl.ds(base, SEG_ROWS)], p0v, sems.at[0]),
            pltpu.async_copy(p_hbm.at[1, pl.ds(base, SEG_ROWS)], p1v, sems.at[1]),
            pltpu.async_copy(h_hbm.at[pl.ds(base, SEG_ROWS)], hv, sems.at[2]),
            pltpu.async_copy(dinv_hbm.at[pl.ds(base, SEG_ROWS)], dv, sems.at[3]),
            pltpu.async_copy(batch_hbm.at[pl.ds(base, SEG_ROWS)], bv, sems.at[4]),
            pltpu.async_copy(b2_hbm, b2v, sems.at[5]),
        ]

        neg16 = jnp.full((16,), NEG, jnp.float32)

        def init_body(i, _):
            pooled[pl.ds(i * 16, 16)] = neg16
            return 0

        lax.fori_loop(0, GP * DH // 16, init_body, 0)
        for cp in cps:
            cp.wait()

        iota16 = lax.iota(jnp.int32, 16)

        def row_body(n, _):
            nvec = jnp.full((16,), n, jnp.int32)
            bvec = plsc.load_gather(bv, [nvec])
            dvec = plsc.load_gather(dv, [nvec])
            pbase = bvec * DH + iota16
            for j in range(DH // 16):
                sl = pl.ds(j * 16, 16)
                val = dvec * (
                    p0v[n, sl] + p1v[n, sl] + hv[n, sl]
                ) + b2v[sl]
                idx = pbase + j * 16
                cur = plsc.load_gather(pooled, [idx])
                plsc.store_scatter(pooled, [idx], jnp.maximum(cur, val))
            return 0

        lax.fori_loop(0, SEG_ROWS, row_body, 0)
        pltpu.sync_copy(pooled, out_hbm.at[w])

    return k(
        p2.reshape(2, NP * DH).reshape(2, NP, DH),
        h2s,
        dinv1d,
        b2,
        batch_p,
    )


def _tc1_body(x_ref, w_ref, deg_ref, h_ref, dinv_ref):
    deg = jnp.sum(deg_ref[...], axis=0) + 1.0
    dinv = lax.rsqrt(deg)
    h = jnp.dot(x_ref[...], w_ref[...], preferred_element_type=jnp.float32)
    h_ref[...] = h * dinv[:, None]
    dinv_ref[...] = dinv[:, None]


def _tc1(xp, W1, degp):
    bn = 1024
    return pl.pallas_call(
        _tc1_body,
        grid=(NP // bn,),
        in_specs=[
            pl.BlockSpec((bn, D_IN), lambda i: (i, 0)),
            pl.BlockSpec((D_IN, DH), lambda i: (0, 0)),
            pl.BlockSpec((NW, bn), lambda i: (0, i)),
        ],
        out_specs=[
            pl.BlockSpec((bn, DH), lambda i: (i, 0)),
            pl.BlockSpec((bn, 1), lambda i: (i, 0)),
        ],
        out_shape=[
            jax.ShapeDtypeStruct((NP, DH), jnp.float32),
            jax.ShapeDtypeStruct((NP, 1), jnp.float32),
        ],
    )(xp, W1, degp)


def _tc2_body(p_ref, h_ref, dinv_ref, b1_ref, w2_ref, out_ref):
    acc = p_ref[0] + p_ref[1]
    dinv = dinv_ref[...]
    t = jnp.maximum(dinv * (acc + h_ref[...]) + b1_ref[...], 0.0)
    out_ref[...] = (
        jnp.dot(t, w2_ref[...], preferred_element_type=jnp.float32) * dinv
    )


def _tc2(p1, h1s, dinv, b1row, W2):
    bn = 1024
    return pl.pallas_call(
        _tc2_body,
        grid=(NP // bn,),
        in_specs=[
            pl.BlockSpec((2, bn, DH), lambda i: (0, i, 0)),
            pl.BlockSpec((bn, DH), lambda i: (i, 0)),
            pl.BlockSpec((bn, 1), lambda i: (i, 0)),
            pl.BlockSpec((1, DH), lambda i: (0, 0)),
            pl.BlockSpec((DH, DH), lambda i: (0, 0)),
        ],
        out_specs=pl.BlockSpec((bn, DH), lambda i: (i, 0)),
        out_shape=jax.ShapeDtypeStruct((NP, DH), jnp.float32),
    )(p1, h1s, dinv, b1row, W2)


def _tc3_body(part_ref, wc_ref, bc_ref, out_ref):
    parts = part_ref[...]
    pooled = jnp.max(parts[:, :G, :], axis=0)
    logits = (
        jnp.dot(pooled, wc_ref[...], preferred_element_type=jnp.float32)
        + bc_ref[...]
    )
    m = jnp.max(logits, axis=1, keepdims=True)
    lse = jnp.log(jnp.sum(jnp.exp(logits - m), axis=1, keepdims=True)) + m
    out_ref[...] = logits - lse


def _tc3(parts, Wc, bcrow):
    nc = bcrow.shape[-1]
    return pl.pallas_call(
        _tc3_body,
        in_specs=[
            pl.BlockSpec((NW, GP, DH), lambda: (0, 0, 0)),
            pl.BlockSpec((DH, nc), lambda: (0, 0)),
            pl.BlockSpec((1, nc), lambda: (0, 0)),
        ],
        out_specs=pl.BlockSpec((G, nc), lambda: (0, 0)),
        out_shape=jax.ShapeDtypeStruct((G, nc), jnp.float32),
    )(parts, Wc, bcrow)


def kernel(x, edge_index, batch, W1, b1, W2, b2, Wc, bc):
    src = edge_index[0].astype(jnp.int32)
    dst = edge_index[1].astype(jnp.int32)
    pad = jnp.full((EP - E,), N_NODES, jnp.int32)
    src2d = jnp.concatenate([src, pad]).reshape(EROWS, CHUNK)
    dst2d = jnp.concatenate([dst, pad]).reshape(EROWS, CHUNK)
    xp = jnp.pad(x, ((0, NP - N_NODES), (0, 0)))
    batch_p = jnp.concatenate(
        [batch.astype(jnp.int32), jnp.full((NP - N_NODES,), G, jnp.int32)]
    )

    degp = _sc_deg(dst2d)
    h1s, dinv = _tc1(xp, W1, degp)
    p1 = _sc_agg(h1s, src2d, dst2d)
    h2s = _tc2(p1, h1s, dinv, b1.reshape(1, DH), W2)
    p2 = _sc_agg(h2s, src2d, dst2d)
    parts = _sc_segmax(p2, h2s, dinv.reshape(NP), b2, batch_p)
    return _tc3(parts.reshape(NW, GP, DH), Wc, bc.reshape(1, -1))


# speedup vs baseline: 18.5052x; 1.0340x over previous
import functools

import jax
import jax.numpy as jnp
from jax import lax
from jax.experimental import pallas as pl
from jax.experimental.pallas import tpu as pltpu
from jax.experimental.pallas import tpu_sc as plsc

N_NODES = 10000
NP = 10240
E = 320000
CHUNK = 128
EROWS = 2560
EP = EROWS * CHUNK
NW = 32
ROWS_PER_W = EROWS // NW
NB = 8
LOOKAHEAD = 4
CH = 64
ECH = EP // CH
CH0 = 256
CH1 = 64
D_IN = 128
DH = 64
G = 64
GP = G + 1
SEG_ROWS = NP // NW
ACC_ROWS = NP // 16
NEG = -3.4028235e38


def _mesh():
    return plsc.VectorSubcoreMesh(
        core_axis_name="c", subcore_axis_name="s", num_cores=2, num_subcores=16
    )


def _sc_deg(dst2d):
    @functools.partial(
        pl.kernel,
        out_type=jax.ShapeDtypeStruct((NW, NP), jnp.float32),
        mesh=_mesh(),
        scratch_types=[
            pltpu.VMEM((ROWS_PER_W, CHUNK), jnp.int32),
            pltpu.VMEM((NP,), jnp.float32),
            pltpu.SemaphoreType.DMA,
        ],
        compiler_params=pltpu.CompilerParams(needs_layout_passes=False),
    )
    def k(dst_hbm, deg_hbm, idx_v, deg_v, sem):
        c = lax.axis_index("c")
        s = lax.axis_index("s")
        w = s * 2 + c
        cp = pltpu.async_copy(
            dst_hbm.at[pl.ds(w * ROWS_PER_W, ROWS_PER_W)], idx_v, sem
        )
        z16 = jnp.zeros((16,), jnp.float32)

        def zero_body(i, _):
            deg_v[pl.ds(i * 16, 16)] = z16
            return 0

        lax.fori_loop(0, NP // 16, zero_body, 0)
        cp.wait()

        def vec_body(j, _):
            idx = idx_v[j // (CHUNK // 16), pl.ds((j % (CHUNK // 16)) * 16, 16)]
            cnt, last = plsc.scan_count(idx)
            plsc.addupdate_scatter(
                deg_v, [idx], cnt.astype(jnp.float32), mask=last
            )
            return 0

        lax.fori_loop(0, ROWS_PER_W * (CHUNK // 16), vec_body, 0)
        pltpu.sync_copy(deg_v, deg_hbm.at[w])

    return k(dst2d)


def _sc_agg(h, src2d, dst2d):
    @functools.partial(
        pl.kernel,
        out_type=jax.ShapeDtypeStruct((2, NP, DH), jnp.float32),
        mesh=_mesh(),
        scratch_types=[
            pltpu.VMEM((CH0, CH), jnp.int32),
            pltpu.VMEM((CH0, CH), jnp.int32),
            pltpu.VMEM((NB, CH, DH), jnp.float32),
            pltpu.VMEM_SHARED((NP, DH), jnp.float32),
            pltpu.SemaphoreType.DMA((NB,)),
            pltpu.SemaphoreType.DMA((NB,)),
            pltpu.SemaphoreType.DMA((2,)),
        ],
        compiler_params=pltpu.CompilerParams(use_tc_tiling_on_sc=False),
    )
    def k(h_hbm, src_hbm, dst_hbm, out_hbm, src_v, dst_v, gbuf, acc_sh,
          sem_g, sem_s, sem_i):
        c = lax.axis_index("c")
        s = lax.axis_index("s")
        z16 = jnp.zeros((16,), jnp.float32)

        def zero_body(i, _):
            for jj in range(DH // 16):
                gbuf[0, i, pl.ds(jj * 16, 16)] = z16
            return 0

        lax.fori_loop(0, CH, zero_body, 0)
        for t in range(ACC_ROWS // CH):
            pltpu.sync_copy(
                gbuf.at[0], acc_sh.at[pl.ds(s * ACC_ROWS + t * CH, CH)]
            )
        plsc.subcore_barrier()

        def span(base, nchunks):
            cps = pltpu.async_copy(
                src_hbm.at[pl.ds(base, nchunks)],
                src_v.at[pl.ds(0, nchunks)], sem_i.at[0],
            )
            cpd = pltpu.async_copy(
                dst_hbm.at[pl.ds(base, nchunks)],
                dst_v.at[pl.ds(0, nchunks)], sem_i.at[1],
            )
            cps.wait()
            cpd.wait()
            for p in range(LOOKAHEAD):
                pltpu.async_copy(
                    h_hbm.at[src_v.at[p]], gbuf.at[p], sem_g.at[p]
                )

            def round_body(r, _):
                for par in range(NB):
                    cidx = r * NB + par
                    pltpu.make_async_copy(
                        h_hbm.at[src_v.at[cidx]], gbuf.at[par], sem_g.at[par]
                    ).wait()
                    pltpu.async_copy(
                        gbuf.at[par], acc_sh.at[dst_v.at[cidx]],
                        sem_s.at[par], add=True,
                    )
                    nslot = (par + LOOKAHEAD) % NB
                    nxt = cidx + LOOKAHEAD

                    @pl.when(nxt >= NB)
                    def _(nxt=nxt, nslot=nslot):
                        pltpu.make_async_copy(
                            gbuf.at[nslot], acc_sh.at[dst_v.at[nxt - NB]],
                            sem_s.at[nslot],
                        ).wait()

                    @pl.when(nxt < nchunks)
                    def _(nxt=nxt, nslot=nslot):
                        pltpu.async_copy(
                            h_hbm.at[src_v.at[nxt]], gbuf.at[nslot],
                            sem_g.at[nslot],
                        )
                return 0

            lax.fori_loop(0, nchunks // NB, round_body, 0)
            for p in range(NB - LOOKAHEAD, NB):
                pltpu.make_async_copy(
                    gbuf.at[p],
                    acc_sh.at[dst_v.at[nchunks - NB + p]],
                    sem_s.at[p],
                ).wait()

        @pl.when(c == 0)
        def _():
            span(s * CH0, CH0)

        @pl.when(c == 1)
        def _():
            span(16 * CH0 + s * CH1, CH1)

        plsc.subcore_barrier()
        pltpu.sync_copy(
            acc_sh.at[pl.ds(s * ACC_ROWS, ACC_ROWS)],
            out_hbm.at[c, pl.ds(s * ACC_ROWS, ACC_ROWS)],
        )

    return k(h, src2d, dst2d)


def _sc_segmax(p2, h2s, dinv1d, b2, batch_p):
    @functools.partial(
        pl.kernel,
        out_type=jax.ShapeDtypeStruct((NW, GP * DH), jnp.float32),
        mesh=_mesh(),
        scratch_types=[
            pltpu.VMEM((SEG_ROWS, DH), jnp.float32),
            pltpu.VMEM((SEG_ROWS, DH), jnp.float32),
            pltpu.VMEM((SEG_ROWS, DH), jnp.float32),
            pltpu.VMEM((SEG_ROWS,), jnp.float32),
            pltpu.VMEM((SEG_ROWS,), jnp.int32),
            pltpu.VMEM((DH,), jnp.float32),
            pltpu.VMEM((GP * DH,), jnp.float32),
            pltpu.SemaphoreType.DMA((6,)),
        ],
        compiler_params=pltpu.CompilerParams(needs_layout_passes=False),
    )
    def k(p_hbm, h_hbm, dinv_hbm, b2_hbm, batch_hbm, out_hbm,
          p0v, p1v, hv, dv, bv, b2v, pooled, sems):
        c = lax.axis_index("c")
        s = lax.axis_index("s")
        w = s * 2 + c
        base = w * SEG_ROWS
        cps = [
            pltpu.async_copy(p_hbm.at[0, pl.ds(base, SEG_ROWS)], p0v, sems.at[0]),
            pltpu.async_copy(p_hbm.at[1, pl.ds(base, SEG_ROWS)], p1v, sems.at[1]),
            pltpu.async_copy(h_hbm.at[pl.ds(base, SEG_ROWS)], hv, sems.at[2]),
            pltpu.async_copy(dinv_hbm.at[pl.ds(base, SEG_ROWS)], dv, sems.at[3]),
            pltpu.async_copy(batch_hbm.at[pl.ds(base, SEG_ROWS)], bv, sems.at[4]),
            pltpu.async_copy(b2_hbm, b2v, sems.at[5]),
        ]

        neg16 = jnp.full((16,), NEG, jnp.float32)

        def init_body(i, _):
            pooled[pl.ds(i * 16, 16)] = neg16
            return 0

        lax.fori_loop(0, GP * DH // 16, init_body, 0)
        for cp in cps:
            cp.wait()

        iota16 = lax.iota(jnp.int32, 16)

        def row_body(n, _):
            nvec = jnp.full((16,), n, jnp.int32)
            bvec = plsc.load_gather(bv, [nvec])
            dvec = plsc.load_gather(dv, [nvec])
            pbase = bvec * DH + iota16
            for j in range(DH // 16):
                sl = pl.ds(j * 16, 16)
                val = dvec * (
                    p0v[n, sl] + p1v[n, sl] + hv[n, sl]
                ) + b2v[sl]
                idx = pbase + j * 16
                cur = plsc.load_gather(pooled, [idx])
                plsc.store_scatter(pooled, [idx], jnp.maximum(cur, val))
            return 0

        lax.fori_loop(0, SEG_ROWS, row_body, 0)
        pltpu.sync_copy(pooled, out_hbm.at[w])

    return k(
        p2.reshape(2, NP * DH).reshape(2, NP, DH),
        h2s,
        dinv1d,
        b2,
        batch_p,
    )


def _tc1_body(x_ref, w_ref, deg_ref, h_ref, dinv_ref):
    deg = jnp.sum(deg_ref[...], axis=0) + 1.0
    dinv = lax.rsqrt(deg)
    h = jnp.dot(x_ref[...], w_ref[...], preferred_element_type=jnp.float32)
    h_ref[...] = h * dinv[:, None]
    dinv_ref[...] = dinv[:, None]


def _tc1(xp, W1, degp):
    bn = 1024
    return pl.pallas_call(
        _tc1_body,
        grid=(NP // bn,),
        in_specs=[
            pl.BlockSpec((bn, D_IN), lambda i: (i, 0)),
            pl.BlockSpec((D_IN, DH), lambda i: (0, 0)),
            pl.BlockSpec((NW, bn), lambda i: (0, i)),
        ],
        out_specs=[
            pl.BlockSpec((bn, DH), lambda i: (i, 0)),
            pl.BlockSpec((bn, 1), lambda i: (i, 0)),
        ],
        out_shape=[
            jax.ShapeDtypeStruct((NP, DH), jnp.float32),
            jax.ShapeDtypeStruct((NP, 1), jnp.float32),
        ],
    )(xp, W1, degp)


def _tc2_body(p_ref, h_ref, dinv_ref, b1_ref, w2_ref, out_ref):
    acc = p_ref[0] + p_ref[1]
    dinv = dinv_ref[...]
    t = jnp.maximum(dinv * (acc + h_ref[...]) + b1_ref[...], 0.0)
    out_ref[...] = (
        jnp.dot(t, w2_ref[...], preferred_element_type=jnp.float32) * dinv
    )


def _tc2(p1, h1s, dinv, b1row, W2):
    bn = 1024
    return pl.pallas_call(
        _tc2_body,
        grid=(NP // bn,),
        in_specs=[
            pl.BlockSpec((2, bn, DH), lambda i: (0, i, 0)),
            pl.BlockSpec((bn, DH), lambda i: (i, 0)),
            pl.BlockSpec((bn, 1), lambda i: (i, 0)),
            pl.BlockSpec((1, DH), lambda i: (0, 0)),
            pl.BlockSpec((DH, DH), lambda i: (0, 0)),
        ],
        out_specs=pl.BlockSpec((bn, DH), lambda i: (i, 0)),
        out_shape=jax.ShapeDtypeStruct((NP, DH), jnp.float32),
    )(p1, h1s, dinv, b1row, W2)


def _tc3_body(part_ref, wc_ref, bc_ref, out_ref):
    parts = part_ref[...]
    pooled = jnp.max(parts[:, :G, :], axis=0)
    logits = (
        jnp.dot(pooled, wc_ref[...], preferred_element_type=jnp.float32)
        + bc_ref[...]
    )
    m = jnp.max(logits, axis=1, keepdims=True)
    lse = jnp.log(jnp.sum(jnp.exp(logits - m), axis=1, keepdims=True)) + m
    out_ref[...] = logits - lse


def _tc3(parts, Wc, bcrow):
    nc = bcrow.shape[-1]
    return pl.pallas_call(
        _tc3_body,
        in_specs=[
            pl.BlockSpec((NW, GP, DH), lambda: (0, 0, 0)),
            pl.BlockSpec((DH, nc), lambda: (0, 0)),
            pl.BlockSpec((1, nc), lambda: (0, 0)),
        ],
        out_specs=pl.BlockSpec((G, nc), lambda: (0, 0)),
        out_shape=jax.ShapeDtypeStruct((G, nc), jnp.float32),
    )(parts, Wc, bcrow)


def kernel(x, edge_index, batch, W1, b1, W2, b2, Wc, bc):
    src = edge_index[0].astype(jnp.int32)
    dst = edge_index[1].astype(jnp.int32)
    pad = jnp.full((EP - E,), N_NODES, jnp.int32)
    src_p = jnp.concatenate([src, pad])
    dst_p = jnp.concatenate([dst, pad])
    dst2d = dst_p.reshape(EROWS, CHUNK)
    src64 = src_p.reshape(ECH, CH)
    dst64 = dst_p.reshape(ECH, CH)
    xp = jnp.pad(x, ((0, NP - N_NODES), (0, 0)))
    batch_p = jnp.concatenate(
        [batch.astype(jnp.int32), jnp.full((NP - N_NODES,), G, jnp.int32)]
    )

    degp = _sc_deg(dst2d)
    h1s, dinv = _tc1(xp, W1, degp)
    p1 = _sc_agg(h1s, src64, dst64)
    h2s = _tc2(p1, h1s, dinv, b1.reshape(1, DH), W2)
    p2 = _sc_agg(h2s, src64, dst64)
    parts = _sc_segmax(p2, h2s, dinv.reshape(NP), b2, batch_p)
    return _tc3(parts.reshape(NW, GP, DH), Wc, bc.reshape(1, -1))


# speedup vs baseline: 18.5273x; 1.0012x over previous
import functools

import jax
import jax.numpy as jnp
from jax import lax
from jax.experimental import pallas as pl
from jax.experimental.pallas import tpu as pltpu
from jax.experimental.pallas import tpu_sc as plsc

N_NODES = 10000
NP = 10240
E = 320000
CHUNK = 128
EROWS = 2560
EP = EROWS * CHUNK
NW = 32
ROWS_PER_W = EROWS // NW
NB = 4
LOOKAHEAD = 2
CH0 = 128
CH1 = 32
D_IN = 128
DH = 64
G = 64
GP = G + 1
SEG_ROWS = NP // NW
ACC_ROWS = NP // 16
NEG = -3.4028235e38


def _mesh():
    return plsc.VectorSubcoreMesh(
        core_axis_name="c", subcore_axis_name="s", num_cores=2, num_subcores=16
    )


def _sc_deg(dst2d):
    @functools.partial(
        pl.kernel,
        out_type=jax.ShapeDtypeStruct((NW, NP), jnp.float32),
        mesh=_mesh(),
        scratch_types=[
            pltpu.VMEM((ROWS_PER_W, CHUNK), jnp.int32),
            pltpu.VMEM((NP,), jnp.float32),
            pltpu.SemaphoreType.DMA,
        ],
        compiler_params=pltpu.CompilerParams(needs_layout_passes=False),
    )
    def k(dst_hbm, deg_hbm, idx_v, deg_v, sem):
        c = lax.axis_index("c")
        s = lax.axis_index("s")
        w = s * 2 + c
        cp = pltpu.async_copy(
            dst_hbm.at[pl.ds(w * ROWS_PER_W, ROWS_PER_W)], idx_v, sem
        )
        z16 = jnp.zeros((16,), jnp.float32)

        def zero_body(i, _):
            deg_v[pl.ds(i * 16, 16)] = z16
            return 0

        lax.fori_loop(0, NP // 16, zero_body, 0)
        cp.wait()

        def vec_body(j, _):
            idx = idx_v[j // (CHUNK // 16), pl.ds((j % (CHUNK // 16)) * 16, 16)]
            cnt, last = plsc.scan_count(idx)
            plsc.addupdate_scatter(
                deg_v, [idx], cnt.astype(jnp.float32), mask=last
            )
            return 0

        lax.fori_loop(0, ROWS_PER_W * (CHUNK // 16), vec_body, 0)
        pltpu.sync_copy(deg_v, deg_hbm.at[w])

    return k(dst2d)


def _sc_agg(h, src2d, dst2d):
    @functools.partial(
        pl.kernel,
        out_type=jax.ShapeDtypeStruct((2, NP, DH), jnp.float32),
        mesh=_mesh(),
        scratch_types=[
            pltpu.VMEM((CH0, CHUNK), jnp.int32),
            pltpu.VMEM((CH0, CHUNK), jnp.int32),
            pltpu.VMEM((NB, CHUNK, DH), jnp.float32),
            pltpu.VMEM_SHARED((NP, DH), jnp.float32),
            pltpu.SemaphoreType.DMA((NB,)),
            pltpu.SemaphoreType.DMA((NB,)),
            pltpu.SemaphoreType.DMA((2,)),
        ],
        compiler_params=pltpu.CompilerParams(use_tc_tiling_on_sc=False),
    )
    def k(h_hbm, src_hbm, dst_hbm, out_hbm, src_v, dst_v, gbuf, acc_sh,
          sem_g, sem_s, sem_i):
        c = lax.axis_index("c")
        s = lax.axis_index("s")
        z16 = jnp.zeros((16,), jnp.float32)

        def zero_body(i, _):
            for jj in range(DH // 16):
                gbuf[0, i, pl.ds(jj * 16, 16)] = z16
            return 0

        lax.fori_loop(0, CHUNK, zero_body, 0)
        for t in range(ACC_ROWS // CHUNK):
            pltpu.sync_copy(
                gbuf.at[0], acc_sh.at[pl.ds(s * ACC_ROWS + t * CHUNK, CHUNK)]
            )
        plsc.subcore_barrier()

        def span(base, nchunks):
            cps = pltpu.async_copy(
                src_hbm.at[pl.ds(base, nchunks)],
                src_v.at[pl.ds(0, nchunks)], sem_i.at[0],
            )
            cpd = pltpu.async_copy(
                dst_hbm.at[pl.ds(base, nchunks)],
                dst_v.at[pl.ds(0, nchunks)], sem_i.at[1],
            )
            cps.wait()
            cpd.wait()
            for p in range(LOOKAHEAD):
                pltpu.async_copy(
                    h_hbm.at[src_v.at[p]], gbuf.at[p], sem_g.at[p]
                )

            def round_body(r, _):
                for par in range(NB):
                    cidx = r * NB + par
                    pltpu.make_async_copy(
                        h_hbm.at[src_v.at[cidx]], gbuf.at[par], sem_g.at[par]
                    ).wait()
                    pltpu.async_copy(
                        gbuf.at[par], acc_sh.at[dst_v.at[cidx]],
                        sem_s.at[par], add=True,
                    )
                    nslot = (par + LOOKAHEAD) % NB
                    nxt = cidx + LOOKAHEAD

                    @pl.when(nxt >= NB)
                    def _(nxt=nxt, nslot=nslot):
                        pltpu.make_async_copy(
                            gbuf.at[nslot], acc_sh.at[dst_v.at[nxt - NB]],
                            sem_s.at[nslot],
                        ).wait()

                    @pl.when(nxt < nchunks)
                    def _(nxt=nxt, nslot=nslot):
                        pltpu.async_copy(
                            h_hbm.at[src_v.at[nxt]], gbuf.at[nslot],
                            sem_g.at[nslot],
                        )
                return 0

            lax.fori_loop(0, nchunks // NB, round_body, 0)
            for p in range(NB - LOOKAHEAD, NB):
                pltpu.make_async_copy(
                    gbuf.at[p],
                    acc_sh.at[dst_v.at[nchunks - NB + p]],
                    sem_s.at[p],
                ).wait()

        @pl.when(c == 0)
        def _():
            span(s * CH0, CH0)

        @pl.when(c == 1)
        def _():
            span(16 * CH0 + s * CH1, CH1)

        plsc.subcore_barrier()
        pltpu.sync_copy(
            acc_sh.at[pl.ds(s * ACC_ROWS, ACC_ROWS)],
            out_hbm.at[c, pl.ds(s * ACC_ROWS, ACC_ROWS)],
        )

    return k(h, src2d, dst2d)


def _sc_segmax(p2, h2s, dinv1d, b2, batch_p):
    @functools.partial(
        pl.kernel,
        out_type=jax.ShapeDtypeStruct((NW, GP * DH), jnp.float32),
        mesh=_mesh(),
        scratch_types=[
            pltpu.VMEM((SEG_ROWS, DH), jnp.float32),
            pltpu.VMEM((SEG_ROWS, DH), jnp.float32),
            pltpu.VMEM((SEG_ROWS, DH), jnp.float32),
            pltpu.VMEM((SEG_ROWS,), jnp.float32),
            pltpu.VMEM((SEG_ROWS,), jnp.int32),
            pltpu.VMEM((DH,), jnp.float32),
            pltpu.VMEM((GP * DH,), jnp.float32),
            pltpu.SemaphoreType.DMA((6,)),
        ],
        compiler_params=pltpu.CompilerParams(needs_layout_passes=False),
    )
    def k(p_hbm, h_hbm, dinv_hbm, b2_hbm, batch_hbm, out_hbm,
          p0v, p1v, hv, dv, bv, b2v, pooled, sems):
        c = lax.axis_index("c")
        s = lax.axis_index("s")
        w = s * 2 + c
        base = w * SEG_ROWS
        cps = [
            pltpu.async_copy(p_hbm.at[0, pl.ds(base, SEG_ROWS)], p0v, sems.at[0]),
            pltpu.async_copy(p_hbm.at[1, pl.ds(base, SEG_ROWS)], p1v, sems.at[1]),
            pltpu.async_copy(h_hbm.at[pl.ds(base, SEG_ROWS)], hv, sems.at[2]),
            pltpu.async_copy(dinv_hbm.at[pl.ds(base, SEG_ROWS)], dv, sems.at[3]),
            pltpu.async_copy(batch_hbm.at[pl.ds(base, SEG_ROWS)], bv, sems.at[4]),
            pltpu.async_copy(b2_hbm, b2v, sems.at[5]),
        ]

        neg16 = jnp.full((16,), NEG, jnp.float32)

        def init_body(i, _):
            pooled[pl.ds(i * 16, 16)] = neg16
            return 0

        lax.fori_loop(0, GP * DH // 16, init_body, 0)
        for cp in cps:
            cp.wait()

        iota16 = lax.iota(jnp.int32, 16)

        def row_body(n, _):
            nvec = jnp.full((16,), n, jnp.int32)
            bvec = plsc.load_gather(bv, [nvec])
            dvec = plsc.load_gather(dv, [nvec])
            pbase = bvec * DH + iota16
            for j in range(DH // 16):
                sl = pl.ds(j * 16, 16)
                val = dvec * (
                    p0v[n, sl] + p1v[n, sl] + hv[n, sl]
                ) + b2v[sl]
                idx = pbase + j * 16
                cur = plsc.load_gather(pooled, [idx])
                plsc.store_scatter(pooled, [idx], jnp.maximum(cur, val))
            return 0

        lax.fori_loop(0, SEG_ROWS, row_body, 0)
        pltpu.sync_copy(pooled, out_hbm.at[w])

    return k(
        p2.reshape(2, NP * DH).reshape(2, NP, DH),
        h2s,
        dinv1d,
        b2,
        batch_p,
    )


def _tc1_body(x_ref, w_ref, deg_ref, h_ref, dinv_ref):
    deg = jnp.sum(deg_ref[...], axis=0) + 1.0
    dinv = lax.rsqrt(deg)
    h = jnp.dot(x_ref[...], w_ref[...], preferred_element_type=jnp.float32)
    h_ref[...] = h * dinv[:, None]
    dinv_ref[...] = dinv[:, None]


def _tc1(xp, W1, degp):
    bn = 1024
    return pl.pallas_call(
        _tc1_body,
        grid=(NP // bn,),
        in_specs=[
            pl.BlockSpec((bn, D_IN), lambda i: (i, 0)),
            pl.BlockSpec((D_IN, DH), lambda i: (0, 0)),
            pl.BlockSpec((NW, bn), lambda i: (0, i)),
        ],
        out_specs=[
            pl.BlockSpec((bn, DH), lambda i: (i, 0)),
            pl.BlockSpec((bn, 1), lambda i: (i, 0)),
        ],
        out_shape=[
            jax.ShapeDtypeStruct((NP, DH), jnp.float32),
            jax.ShapeDtypeStruct((NP, 1), jnp.float32),
        ],
    )(xp, W1, degp)


def _tc2_body(p_ref, h_ref, dinv_ref, b1_ref, w2_ref, out_ref):
    acc = p_ref[0] + p_ref[1]
    dinv = dinv_ref[...]
    t = jnp.maximum(dinv * (acc + h_ref[...]) + b1_ref[...], 0.0)
    out_ref[...] = (
        jnp.dot(t, w2_ref[...], preferred_element_type=jnp.float32) * dinv
    )


def _tc2(p1, h1s, dinv, b1row, W2):
    bn = 1024
    return pl.pallas_call(
        _tc2_body,
        grid=(NP // bn,),
        in_specs=[
            pl.BlockSpec((2, bn, DH), lambda i: (0, i, 0)),
            pl.BlockSpec((bn, DH), lambda i: (i, 0)),
            pl.BlockSpec((bn, 1), lambda i: (i, 0)),
            pl.BlockSpec((1, DH), lambda i: (0, 0)),
            pl.BlockSpec((DH, DH), lambda i: (0, 0)),
        ],
        out_specs=pl.BlockSpec((bn, DH), lambda i: (i, 0)),
        out_shape=jax.ShapeDtypeStruct((NP, DH), jnp.float32),
    )(p1, h1s, dinv, b1row, W2)


def _tc3_body(part_ref, wc_ref, bc_ref, out_ref):
    parts = part_ref[...]
    pooled = jnp.max(parts[:, :G, :], axis=0)
    logits = (
        jnp.dot(pooled, wc_ref[...], preferred_element_type=jnp.float32)
        + bc_ref[...]
    )
    m = jnp.max(logits, axis=1, keepdims=True)
    lse = jnp.log(jnp.sum(jnp.exp(logits - m), axis=1, keepdims=True)) + m
    out_ref[...] = logits - lse


def _tc3(parts, Wc, bcrow):
    nc = bcrow.shape[-1]
    return pl.pallas_call(
        _tc3_body,
        in_specs=[
            pl.BlockSpec((NW, GP, DH), lambda: (0, 0, 0)),
            pl.BlockSpec((DH, nc), lambda: (0, 0)),
            pl.BlockSpec((1, nc), lambda: (0, 0)),
        ],
        out_specs=pl.BlockSpec((G, nc), lambda: (0, 0)),
        out_shape=jax.ShapeDtypeStruct((G, nc), jnp.float32),
    )(parts, Wc, bcrow)


def kernel(x, edge_index, batch, W1, b1, W2, b2, Wc, bc):
    src = edge_index[0].astype(jnp.int32)
    dst = edge_index[1].astype(jnp.int32)
    pad = jnp.full((EP - E,), N_NODES, jnp.int32)
    src2d = jnp.concatenate([src, pad]).reshape(EROWS, CHUNK)
    dst2d = jnp.concatenate([dst, pad]).reshape(EROWS, CHUNK)
    xp = jnp.pad(x, ((0, NP - N_NODES), (0, 0)))
    batch_p = jnp.concatenate(
        [batch.astype(jnp.int32), jnp.full((NP - N_NODES,), G, jnp.int32)]
    )

    degp = _sc_deg(dst2d)
    h1s, dinv = _tc1(xp, W1, degp)
    p1 = _sc_agg(h1s, src2d, dst2d)
    h2s = _tc2(p1, h1s, dinv, b1.reshape(1, DH), W2)
    p2 = _sc_agg(h2s, src2d, dst2d)
    parts = _sc_segmax(p2, h2s, dinv.reshape(NP), b2, batch_p)
    return _tc3(parts.reshape(NW, GP, DH), Wc, bc.reshape(1, -1))


# speedup vs baseline: 18.5585x; 1.0017x over previous
import functools

import jax
import jax.numpy as jnp
from jax import lax
from jax.experimental import pallas as pl
from jax.experimental.pallas import tpu as pltpu
from jax.experimental.pallas import tpu_sc as plsc

N_NODES = 10000
NP = 10240
E = 320000
CHUNK = 128
EROWS = 2560
EP = EROWS * CHUNK
NW = 32
ROWS_PER_W = EROWS // NW
NB = 4
LOOKAHEAD = 2
CH0 = 128
CH1 = 32
D_IN = 128
DH = 64
G = 64
GP = G + 1
SEG_ROWS = NP // NW
ACC_ROWS = NP // 16
NEG = -3.4028235e38


def _mesh():
    return plsc.VectorSubcoreMesh(
        core_axis_name="c", subcore_axis_name="s", num_cores=2, num_subcores=16
    )


def _sc_deg(dst2d):
    @functools.partial(
        pl.kernel,
        out_type=jax.ShapeDtypeStruct((NW, NP), jnp.float32),
        mesh=_mesh(),
        scratch_types=[
            pltpu.VMEM((ROWS_PER_W, CHUNK), jnp.int32),
            pltpu.VMEM((NP,), jnp.float32),
            pltpu.SemaphoreType.DMA,
        ],
        compiler_params=pltpu.CompilerParams(needs_layout_passes=False),
    )
    def k(dst_hbm, deg_hbm, idx_v, deg_v, sem):
        c = lax.axis_index("c")
        s = lax.axis_index("s")
        w = s * 2 + c
        cp = pltpu.async_copy(
            dst_hbm.at[pl.ds(w * ROWS_PER_W, ROWS_PER_W)], idx_v, sem
        )
        z16 = jnp.zeros((16,), jnp.float32)

        def zero_body(i, _):
            deg_v[pl.ds(i * 16, 16)] = z16
            return 0

        lax.fori_loop(0, NP // 16, zero_body, 0)
        cp.wait()

        def vec_body(j, _):
            idx = idx_v[j // (CHUNK // 16), pl.ds((j % (CHUNK // 16)) * 16, 16)]
            cnt, last = plsc.scan_count(idx)
            plsc.addupdate_scatter(
                deg_v, [idx], cnt.astype(jnp.float32), mask=last
            )
            return 0

        lax.fori_loop(0, ROWS_PER_W * (CHUNK // 16), vec_body, 0)
        pltpu.sync_copy(deg_v, deg_hbm.at[w])

    return k(dst2d)


def _sc_agg(h, src2d, dst2d):
    @functools.partial(
        pl.kernel,
        out_type=jax.ShapeDtypeStruct((2, NP, DH), jnp.float32),
        mesh=_mesh(),
        scratch_types=[
            pltpu.VMEM((CH0, CHUNK), jnp.int32),
            pltpu.VMEM((CH0, CHUNK), jnp.int32),
            pltpu.VMEM((NB, CHUNK, DH), jnp.float32),
            pltpu.VMEM_SHARED((NP, DH), jnp.float32),
            pltpu.SemaphoreType.DMA((NB,)),
            pltpu.SemaphoreType.DMA((NB,)),
            pltpu.SemaphoreType.DMA((2,)),
        ],
        compiler_params=pltpu.CompilerParams(use_tc_tiling_on_sc=False),
    )
    def k(h_hbm, src_hbm, dst_hbm, out_hbm, src_v, dst_v, gbuf, acc_sh,
          sem_g, sem_s, sem_i):
        c = lax.axis_index("c")
        s = lax.axis_index("s")
        z16 = jnp.zeros((16,), jnp.float32)

        def zero_body(i, _):
            for jj in range(DH // 16):
                gbuf[0, i, pl.ds(jj * 16, 16)] = z16
            return 0

        with jax.named_scope("agg_zero"):
            lax.fori_loop(0, CHUNK, zero_body, 0)
            for t in range(ACC_ROWS // CHUNK):
                pltpu.sync_copy(
                    gbuf.at[0],
                    acc_sh.at[pl.ds(s * ACC_ROWS + t * CHUNK, CHUNK)],
                )
            plsc.subcore_barrier()

        def span(base, nchunks):
            cps = pltpu.async_copy(
                src_hbm.at[pl.ds(base, nchunks)],
                src_v.at[pl.ds(0, nchunks)], sem_i.at[0],
            )
            cpd = pltpu.async_copy(
                dst_hbm.at[pl.ds(base, nchunks)],
                dst_v.at[pl.ds(0, nchunks)], sem_i.at[1],
            )
            cps.wait()
            cpd.wait()
            for p in range(LOOKAHEAD):
                pltpu.async_copy(
                    h_hbm.at[src_v.at[p]], gbuf.at[p], sem_g.at[p]
                )

            def round_body(r, _):
                for par in range(NB):
                    cidx = r * NB + par
                    pltpu.make_async_copy(
                        h_hbm.at[src_v.at[cidx]], gbuf.at[par], sem_g.at[par]
                    ).wait()
                    pltpu.async_copy(
                        gbuf.at[par], acc_sh.at[dst_v.at[cidx]],
                        sem_s.at[par], add=True,
                    )
                    nslot = (par + LOOKAHEAD) % NB
                    nxt = cidx + LOOKAHEAD

                    @pl.when(nxt >= NB)
                    def _(nxt=nxt, nslot=nslot):
                        pltpu.make_async_copy(
                            gbuf.at[nslot], acc_sh.at[dst_v.at[nxt - NB]],
                            sem_s.at[nslot],
                        ).wait()

                    @pl.when(nxt < nchunks)
                    def _(nxt=nxt, nslot=nslot):
                        pltpu.async_copy(
                            h_hbm.at[src_v.at[nxt]], gbuf.at[nslot],
                            sem_g.at[nslot],
                        )
                return 0

            lax.fori_loop(0, nchunks // NB, round_body, 0)
            for p in range(NB - LOOKAHEAD, NB):
                pltpu.make_async_copy(
                    gbuf.at[p],
                    acc_sh.at[dst_v.at[nchunks - NB + p]],
                    sem_s.at[p],
                ).wait()

        with jax.named_scope("agg_edges"):
            @pl.when(c == 0)
            def _():
                span(s * CH0, CH0)

            @pl.when(c == 1)
            def _():
                span(16 * CH0 + s * CH1, CH1)

            plsc.subcore_barrier()

        with jax.named_scope("agg_wb"):
            pltpu.sync_copy(
                acc_sh.at[pl.ds(s * ACC_ROWS, ACC_ROWS)],
                out_hbm.at[c, pl.ds(s * ACC_ROWS, ACC_ROWS)],
            )

    return k(h, src2d, dst2d)


def _sc_segmax(p2, h2s, dinv1d, b2, batch_p):
    @functools.partial(
        pl.kernel,
        out_type=jax.ShapeDtypeStruct((NW, GP * DH), jnp.float32),
        mesh=_mesh(),
        scratch_types=[
            pltpu.VMEM((SEG_ROWS, DH), jnp.float32),
            pltpu.VMEM((SEG_ROWS, DH), jnp.float32),
            pltpu.VMEM((SEG_ROWS, DH), jnp.float32),
            pltpu.VMEM((SEG_ROWS,), jnp.float32),
            pltpu.VMEM((SEG_ROWS,), jnp.int32),
            pltpu.VMEM((DH,), jnp.float32),
            pltpu.VMEM((GP * DH,), jnp.float32),
            pltpu.SemaphoreType.DMA((6,)),
        ],
        compiler_params=pltpu.CompilerParams(needs_layout_passes=False),
    )
    def k(p_hbm, h_hbm, dinv_hbm, b2_hbm, batch_hbm, out_hbm,
          p0v, p1v, hv, dv, bv, b2v, pooled, sems):
        c = lax.axis_index("c")
        s = lax.axis_index("s")
        w = s * 2 + c
        base = w * SEG_ROWS
        cps = [
            pltpu.async_copy(p_hbm.at[0, pl.ds(base, SEG_ROWS)], p0v, sems.at[0]),
            pltpu.async_copy(p_hbm.at[1, pl.ds(base, SEG_ROWS)], p1v, sems.at[1]),
            pltpu.async_copy(h_hbm.at[pl.ds(base, SEG_ROWS)], hv, sems.at[2]),
            pltpu.async_copy(dinv_hbm.at[pl.ds(base, SEG_ROWS)], dv, sems.at[3]),
            pltpu.async_copy(batch_hbm.at[pl.ds(base, SEG_ROWS)], bv, sems.at[4]),
            pltpu.async_copy(b2_hbm, b2v, sems.at[5]),
        ]

        neg16 = jnp.full((16,), NEG, jnp.float32)

        def init_body(i, _):
            pooled[pl.ds(i * 16, 16)] = neg16
            return 0

        lax.fori_loop(0, GP * DH // 16, init_body, 0)
        for cp in cps:
            cp.wait()

        iota16 = lax.iota(jnp.int32, 16)

        def row_body(n, _):
            nvec = jnp.full((16,), n, jnp.int32)
            bvec = plsc.load_gather(bv, [nvec])
            dvec = plsc.load_gather(dv, [nvec])
            pbase = bvec * DH + iota16
            for j in range(DH // 16):
                sl = pl.ds(j * 16, 16)
                val = dvec * (
                    p0v[n, sl] + p1v[n, sl] + hv[n, sl]
                ) + b2v[sl]
                idx = pbase + j * 16
                cur = plsc.load_gather(pooled, [idx])
                plsc.store_scatter(pooled, [idx], jnp.maximum(cur, val))
            return 0

        lax.fori_loop(0, SEG_ROWS, row_body, 0)
        pltpu.sync_copy(pooled, out_hbm.at[w])

    return k(
        p2.reshape(2, NP * DH).reshape(2, NP, DH),
        h2s,
        dinv1d,
        b2,
        batch_p,
    )


def _tc1_body(x_ref, w_ref, deg_ref, h_ref, dinv_ref):
    deg = jnp.sum(deg_ref[...], axis=0) + 1.0
    dinv = lax.rsqrt(deg)
    h = jnp.dot(x_ref[...], w_ref[...], preferred_element_type=jnp.float32)
    h_ref[...] = h * dinv[:, None]
    dinv_ref[...] = dinv[:, None]


def _tc1(xp, W1, degp):
    bn = 1024
    return pl.pallas_call(
        _tc1_body,
        grid=(NP // bn,),
        in_specs=[
            pl.BlockSpec((bn, D_IN), lambda i: (i, 0)),
            pl.BlockSpec((D_IN, DH), lambda i: (0, 0)),
            pl.BlockSpec((NW, bn), lambda i: (0, i)),
        ],
        out_specs=[
            pl.BlockSpec((bn, DH), lambda i: (i, 0)),
            pl.BlockSpec((bn, 1), lambda i: (i, 0)),
        ],
        out_shape=[
            jax.ShapeDtypeStruct((NP, DH), jnp.float32),
            jax.ShapeDtypeStruct((NP, 1), jnp.float32),
        ],
    )(xp, W1, degp)


def _tc2_body(p_ref, h_ref, dinv_ref, b1_ref, w2_ref, out_ref):
    acc = p_ref[0] + p_ref[1]
    dinv = dinv_ref[...]
    t = jnp.maximum(dinv * (acc + h_ref[...]) + b1_ref[...], 0.0)
    out_ref[...] = (
        jnp.dot(t, w2_ref[...], preferred_element_type=jnp.float32) * dinv
    )


def _tc2(p1, h1s, dinv, b1row, W2):
    bn = 1024
    return pl.pallas_call(
        _tc2_body,
        grid=(NP // bn,),
        in_specs=[
            pl.BlockSpec((2, bn, DH), lambda i: (0, i, 0)),
            pl.BlockSpec((bn, DH), lambda i: (i, 0)),
            pl.BlockSpec((bn, 1), lambda i: (i, 0)),
            pl.BlockSpec((1, DH), lambda i: (0, 0)),
            pl.BlockSpec((DH, DH), lambda i: (0, 0)),
        ],
        out_specs=pl.BlockSpec((bn, DH), lambda i: (i, 0)),
        out_shape=jax.ShapeDtypeStruct((NP, DH), jnp.float32),
    )(p1, h1s, dinv, b1row, W2)


def _tc3_body(part_ref, wc_ref, bc_ref, out_ref):
    parts = part_ref[...]
    pooled = jnp.max(parts[:, :G, :], axis=0)
    logits = (
        jnp.dot(pooled, wc_ref[...], preferred_element_type=jnp.float32)
        + bc_ref[...]
    )
    m = jnp.max(logits, axis=1, keepdims=True)
    lse = jnp.log(jnp.sum(jnp.exp(logits - m), axis=1, keepdims=True)) + m
    out_ref[...] = logits - lse


def _tc3(parts, Wc, bcrow):
    nc = bcrow.shape[-1]
    return pl.pallas_call(
        _tc3_body,
        in_specs=[
            pl.BlockSpec((NW, GP, DH), lambda: (0, 0, 0)),
            pl.BlockSpec((DH, nc), lambda: (0, 0)),
            pl.BlockSpec((1, nc), lambda: (0, 0)),
        ],
        out_specs=pl.BlockSpec((G, nc), lambda: (0, 0)),
        out_shape=jax.ShapeDtypeStruct((G, nc), jnp.float32),
    )(parts, Wc, bcrow)


def kernel(x, edge_index, batch, W1, b1, W2, b2, Wc, bc):
    src = edge_index[0].astype(jnp.int32)
    dst = edge_index[1].astype(jnp.int32)
    pad = jnp.full((EP - E,), N_NODES, jnp.int32)
    src2d = jnp.concatenate([src, pad]).reshape(EROWS, CHUNK)
    dst2d = jnp.concatenate([dst, pad]).reshape(EROWS, CHUNK)
    xp = jnp.pad(x, ((0, NP - N_NODES), (0, 0)))
    batch_p = jnp.concatenate(
        [batch.astype(jnp.int32), jnp.full((NP - N_NODES,), G, jnp.int32)]
    )

    degp = _sc_deg(dst2d)
    h1s, dinv = _tc1(xp, W1, degp)
    p1 = _sc_agg(h1s, src2d, dst2d)
    h2s = _tc2(p1, h1s, dinv, b1.reshape(1, DH), W2)
    p2 = _sc_agg(h2s, src2d, dst2d)
    parts = _sc_segmax(p2, h2s, dinv.reshape(NP), b2, batch_p)
    return _tc3(parts.reshape(NW, GP, DH), Wc, bc.reshape(1, -1))
